# Initial kernel scaffold; baseline (speedup 1.0000x reference)
import functools

import jax
import jax.numpy as jnp
from jax import lax
from jax.experimental import pallas as pl
from jax.experimental.pallas import tpu as pltpu
from jax.experimental.pallas import tpu_sc as plsc

N = 10000
E = 320000
D = 128
INV_TAU = 2.0

NC = 2
NS = 16
EDGE_CHUNK = 80
EPW = E // NS
NCHUNK = EPW // EDGE_CHUNK
ROWS_PW = N // NS

_MESH = plsc.VectorSubcoreMesh(
    core_axis_name="c", subcore_axis_name="s", num_cores=NC, num_subcores=NS)


@functools.partial(
    pl.kernel,
    out_type=jax.ShapeDtypeStruct((NC, N, 16), jnp.float32),
    mesh=_MESH,
    scratch_types=[
        pltpu.VMEM((EDGE_CHUNK,), jnp.int32),
        pltpu.VMEM((EDGE_CHUNK, 16), jnp.float32),
        pltpu.VMEM_SHARED((N, 16), jnp.float32),
    ],
)
def _deg_kernel(ei_hbm, ones_hbm, zeros_hbm, deg_hbm, idx_v, ones_v, acc_sh):
    g = lax.axis_index("c")
    s = lax.axis_index("s")
    r0 = s * ROWS_PW
    pltpu.sync_copy(zeros_hbm.at[pl.ds(r0, ROWS_PW)], acc_sh.at[pl.ds(r0, ROWS_PW)])
    pltpu.sync_copy(ones_hbm, ones_v)
    plsc.subcore_barrier()

    def body(i, carry):
        e0 = s * EPW + i * EDGE_CHUNK
        pltpu.sync_copy(ei_hbm.at[g, 1, pl.ds(e0, EDGE_CHUNK)], idx_v)
        pltpu.sync_copy(ones_v, acc_sh.at[idx_v], add=True)
        return carry

    lax.fori_loop(0, NCHUNK, body, 0)
    plsc.subcore_barrier()
    pltpu.sync_copy(acc_sh.at[pl.ds(r0, ROWS_PW)], deg_hbm.at[g, pl.ds(r0, ROWS_PW)])


@functools.partial(
    pl.kernel,
    out_type=jax.ShapeDtypeStruct((NC * N, D), jnp.float32),
    mesh=_MESH,
    scratch_types=[
        pltpu.VMEM((EDGE_CHUNK,), jnp.int32),
        pltpu.VMEM((EDGE_CHUNK,), jnp.int32),
        pltpu.VMEM((EDGE_CHUNK, D), jnp.float32),
        pltpu.VMEM_SHARED((N, D), jnp.float32),
        pltpu.SemaphoreType.DMA,
    ],
)
def _agg_kernel(h_hbm, ei_hbm, zeros_hbm, out_hbm, sidx_v, didx_v, rows_v,
                acc_sh, sem):
    g = lax.axis_index("c")
    s = lax.axis_index("s")
    r0 = s * ROWS_PW
    pltpu.sync_copy(zeros_hbm.at[pl.ds(r0, ROWS_PW)], acc_sh.at[pl.ds(r0, ROWS_PW)])
    plsc.subcore_barrier()
    goff = g * N

    def body(i, carry):
        e0 = s * EPW + i * EDGE_CHUNK
        pltpu.sync_copy(ei_hbm.at[g, 0, pl.ds(e0, EDGE_CHUNK)], sidx_v)
        pltpu.sync_copy(ei_hbm.at[g, 1, pl.ds(e0, EDGE_CHUNK)], didx_v)
        for k in range(EDGE_CHUNK // 16):
            sl = pl.ds(k * 16, 16)
            sidx_v[sl] = sidx_v[sl] + jnp.full((16,), goff, jnp.int32)
        pltpu.async_copy(h_hbm.at[sidx_v], rows_v, sem).wait()
        pltpu.sync_copy(rows_v, acc_sh.at[didx_v], add=True)
        return carry

    lax.fori_loop(0, NCHUNK, body, 0)
    plsc.subcore_barrier()
    pltpu.sync_copy(acc_sh.at[pl.ds(r0, ROWS_PW)],
                    out_hbm.at[pl.ds(goff + r0, ROWS_PW)])


_BLK = 1000


def _dinv(deg_blk):
    return lax.rsqrt(jnp.maximum(deg_blk[:, :1], 1.0))


def _k1_body(x_ref, w_ref, b_ref, deg_ref, o_ref):
    h = jnp.dot(x_ref[...], w_ref[...], preferred_element_type=jnp.float32)
    h = h + b_ref[...]
    o_ref[...] = h * _dinv(deg_ref[...])


def _k2_body(a_ref, w_ref, b_ref, deg_ref, o_ref):
    di = _dinv(deg_ref[...])
    h = jnp.maximum(a_ref[...] * di, 0.0)
    h = jnp.dot(h, w_ref[...], preferred_element_type=jnp.float32) + b_ref[...]
    o_ref[...] = h * di


def _k3_body(a_ref, deg_ref, z_ref, zn_ref):
    z = a_ref[...] * _dinv(deg_ref[...])
    z_ref[...] = z
    nrm = jnp.sqrt(jnp.sum(z * z, axis=1, keepdims=True))
    zn_ref[...] = z / (nrm + 1e-8)


def _run_k1(xs, W1, b1, deg16):
    return pl.pallas_call(
        _k1_body,
        grid=((NC * N) // _BLK,),
        in_specs=[
            pl.BlockSpec((_BLK, D), lambda i: (i, 0)),
            pl.BlockSpec((D, D), lambda i: (0, 0)),
            pl.BlockSpec((1, D), lambda i: (0, 0)),
            pl.BlockSpec((_BLK, 16), lambda i: (i, 0)),
        ],
        out_specs=pl.BlockSpec((_BLK, D), lambda i: (i, 0)),
        out_shape=jax.ShapeDtypeStruct((NC * N, D), jnp.float32),
    )(xs, W1, b1, deg16)


def _run_k2(agg, W2, b2, deg16):
    return pl.pallas_call(
        _k2_body,
        grid=((NC * N) // _BLK,),
        in_specs=[
            pl.BlockSpec((_BLK, D), lambda i: (i, 0)),
            pl.BlockSpec((D, D), lambda i: (0, 0)),
            pl.BlockSpec((1, D), lambda i: (0, 0)),
            pl.BlockSpec((_BLK, 16), lambda i: (i, 0)),
        ],
        out_specs=pl.BlockSpec((_BLK, D), lambda i: (i, 0)),
        out_shape=jax.ShapeDtypeStruct((NC * N, D), jnp.float32),
    )(agg, W2, b2, deg16)


def _run_k3(agg, deg16):
    return pl.pallas_call(
        _k3_body,
        grid=((NC * N) // _BLK,),
        in_specs=[
            pl.BlockSpec((_BLK, D), lambda i: (i, 0)),
            pl.BlockSpec((_BLK, 16), lambda i: (i, 0)),
        ],
        out_specs=[
            pl.BlockSpec((_BLK, D), lambda i: (i, 0)),
            pl.BlockSpec((_BLK, D), lambda i: (i, 0)),
        ],
        out_shape=[
            jax.ShapeDtypeStruct((NC * N, D), jnp.float32),
            jax.ShapeDtypeStruct((NC * N, D), jnp.float32),
        ],
    )(agg, deg16)


_LB = 1000
_LG = N // _LB


def _loss_body(z1_ref, z2_ref, o_ref, rs_ref, cs_ref, dg_ref):
    i = pl.program_id(0)
    j = pl.program_id(1)
    sim = jax.lax.dot_general(
        z1_ref[...], z2_ref[...], (((1,), (1,)), ((), ())),
        preferred_element_type=jnp.float32) * INV_TAU
    es = jnp.exp(sim)
    row_s = jnp.sum(es, axis=1, keepdims=True)
    col_s = jnp.sum(es, axis=0, keepdims=True)

    @pl.when(j == 0)
    def _():
        rs_ref[:, pl.ds(i, 1)] = row_s

    @pl.when(j != 0)
    def _():
        rs_ref[:, pl.ds(i, 1)] += row_s

    @pl.when(i == 0)
    def _():
        cs_ref[pl.ds(j, 1), :] = col_s

    @pl.when(i != 0)
    def _():
        cs_ref[pl.ds(j, 1), :] += col_s

    @pl.when(i == j)
    def _():
        r = lax.broadcasted_iota(jnp.int32, (_LB, _LB), 0)
        c = lax.broadcasted_iota(jnp.int32, (_LB, _LB), 1)
        dvals = jnp.sum(jnp.where(r == c, sim, 0.0), axis=1, keepdims=True)
        dg_ref[:, pl.ds(i, 1)] = dvals

    @pl.when((i == _LG - 1) & (j == _LG - 1))
    def _():
        total = (2.0 * jnp.sum(dg_ref[...])
                 - jnp.sum(jnp.log(rs_ref[...]))
                 - jnp.sum(jnp.log(cs_ref[...])))
        o_ref[0, 0] = -0.5 * total / N


def _run_loss(z1n, z2n):
    return pl.pallas_call(
        _loss_body,
        grid=(_LG, _LG),
        in_specs=[
            pl.BlockSpec((_LB, D), lambda i, j: (i, 0)),
            pl.BlockSpec((_LB, D), lambda i, j: (j, 0)),
        ],
        out_specs=pl.BlockSpec((1, 1), lambda i, j: (0, 0)),
        out_shape=jax.ShapeDtypeStruct((1, 1), jnp.float32),
        scratch_shapes=[
            pltpu.VMEM((_LB, _LG), jnp.float32),
            pltpu.VMEM((_LG, _LB), jnp.float32),
            pltpu.VMEM((_LB, _LG), jnp.float32),
        ],
    )(z1n, z2n)


def kernel(x1, edge_index1, x2, edge_index2, W1, b1, W2, b2):
    ei = jnp.stack([edge_index1.astype(jnp.int32),
                    edge_index2.astype(jnp.int32)])
    xs = jnp.concatenate([x1, x2], axis=0)
    ones16 = jnp.ones((EDGE_CHUNK, 16), jnp.float32)
    zeros16 = jnp.zeros((N, 16), jnp.float32)
    zerosD = jnp.zeros((N, D), jnp.float32)
    b1r = b1.reshape(1, D)
    b2r = b2.reshape(1, D)

    deg16 = _deg_kernel(ei, ones16, zeros16)
    deg16 = deg16.reshape(NC * N, 16)

    h1s = _run_k1(xs, W1, b1r, deg16)
    agg1 = _agg_kernel(h1s, ei, zerosD)
    h2s = _run_k2(agg1, W2, b2r, deg16)
    agg2 = _agg_kernel(h2s, ei, zerosD)
    z, zn = _run_k3(agg2, deg16)

    z1 = z[:N]
    loss = _run_loss(zn[:N], zn[N:])[0, 0]
    return (z1, loss)


# speedup vs baseline: 8.2531x; 8.2531x over previous
import jax
import jax.numpy as jnp
from jax import lax
from jax.experimental import pallas as pl
from jax.experimental.pallas import tpu as pltpu
from jax.experimental.pallas import tpu_sc as plsc

N = 10000
NP = 10240
E = 320000
D = 128
INV_TAU = 2.0

NC = 2
NS = 16
EDGE_CHUNK = 80
EPW = E // NS
NCHUNK = EPW // EDGE_CHUNK
ROWS_PW = NP // NS

_MESH = plsc.VectorSubcoreMesh(
    core_axis_name="c", subcore_axis_name="s", num_cores=NC, num_subcores=NS)


def _deg_kernel(dst_hbm, ones_hbm, zeros_hbm, deg_hbm, idx_v, ones_v, acc_sh):
    g = lax.axis_index("c")
    s = lax.axis_index("s")
    r0 = s * ROWS_PW
    pltpu.sync_copy(zeros_hbm.at[pl.ds(r0, ROWS_PW)], acc_sh.at[pl.ds(r0, ROWS_PW)])
    pltpu.sync_copy(ones_hbm, ones_v)
    plsc.subcore_barrier()

    def body(i, carry):
        e0 = g * E + s * EPW + i * EDGE_CHUNK
        pltpu.sync_copy(dst_hbm.at[pl.ds(e0, EDGE_CHUNK)], idx_v)
        pltpu.sync_copy(ones_v, acc_sh.at[idx_v], add=True)
        return carry

    lax.fori_loop(0, NCHUNK, body, 0)
    plsc.subcore_barrier()
    pltpu.sync_copy(acc_sh.at[pl.ds(r0, ROWS_PW)],
                    deg_hbm.at[pl.ds(g * NP + r0, ROWS_PW)])


def _agg_kernel(h_hbm, src_hbm, dst_hbm, zeros_hbm, out_hbm, sidx_v, didx_v,
                rows_v, acc_sh, sem):
    g = lax.axis_index("c")
    s = lax.axis_index("s")
    r0 = s * ROWS_PW
    pltpu.sync_copy(zeros_hbm.at[pl.ds(r0, ROWS_PW)], acc_sh.at[pl.ds(r0, ROWS_PW)])
    plsc.subcore_barrier()
    goff = g * NP

    def body(i, carry):
        e0 = g * E + s * EPW + i * EDGE_CHUNK
        pltpu.sync_copy(src_hbm.at[pl.ds(e0, EDGE_CHUNK)], sidx_v)
        pltpu.sync_copy(dst_hbm.at[pl.ds(e0, EDGE_CHUNK)], didx_v)
        for k in range(EDGE_CHUNK // 16):
            sl = pl.ds(k * 16, 16)
            sidx_v[sl] = sidx_v[sl] + jnp.full((16,), goff, jnp.int32)
        pltpu.async_copy(h_hbm.at[sidx_v], rows_v, sem).wait()
        pltpu.sync_copy(rows_v, acc_sh.at[didx_v], add=True)
        return carry

    lax.fori_loop(0, NCHUNK, body, 0)
    plsc.subcore_barrier()
    pltpu.sync_copy(acc_sh.at[pl.ds(r0, ROWS_PW)],
                    out_hbm.at[pl.ds(goff + r0, ROWS_PW)])


def _make_deg(interpret=False):
    return pl.kernel(
        _deg_kernel,
        out_type=jax.ShapeDtypeStruct((NC * NP, D), jnp.float32),
        mesh=_MESH,
        scratch_types=[
            pltpu.VMEM((EDGE_CHUNK,), jnp.int32),
            pltpu.VMEM((EDGE_CHUNK, D), jnp.float32),
            pltpu.VMEM_SHARED((NP, D), jnp.float32),
        ],
        interpret=interpret,
    )


def _make_agg(interpret=False):
    return pl.kernel(
        _agg_kernel,
        out_type=jax.ShapeDtypeStruct((NC * NP, D), jnp.float32),
        mesh=_MESH,
        scratch_types=[
            pltpu.VMEM((EDGE_CHUNK,), jnp.int32),
            pltpu.VMEM((EDGE_CHUNK,), jnp.int32),
            pltpu.VMEM((EDGE_CHUNK, D), jnp.float32),
            pltpu.VMEM_SHARED((NP, D), jnp.float32),
            pltpu.SemaphoreType.DMA,
        ],
        interpret=interpret,
    )


_deg_call = _make_deg()
_agg_call = _make_agg()


_BLK = 1024


def _dinv(deg_blk):
    return lax.rsqrt(jnp.maximum(deg_blk[:, :1], 1.0))


def _k1_body(x_ref, w_ref, b_ref, deg_ref, o_ref):
    h = jnp.dot(x_ref[...], w_ref[...], preferred_element_type=jnp.float32)
    h = h + b_ref[...]
    o_ref[...] = h * _dinv(deg_ref[...])


def _k2_body(a_ref, w_ref, b_ref, deg_ref, o_ref):
    di = _dinv(deg_ref[...])
    h = jnp.maximum(a_ref[...] * di, 0.0)
    h = jnp.dot(h, w_ref[...], preferred_element_type=jnp.float32) + b_ref[...]
    o_ref[...] = h * di


def _k3_body(a_ref, deg_ref, z_ref, zn_ref):
    z = a_ref[...] * _dinv(deg_ref[...])
    z_ref[...] = z
    nrm = jnp.sqrt(jnp.sum(z * z, axis=1, keepdims=True))
    zn_ref[...] = z / (nrm + 1e-8)


def _run_k1(xs, W1, b1, deg16):
    return pl.pallas_call(
        _k1_body,
        grid=((NC * NP) // _BLK,),
        in_specs=[
            pl.BlockSpec((_BLK, D), lambda i: (i, 0)),
            pl.BlockSpec((D, D), lambda i: (0, 0)),
            pl.BlockSpec((1, D), lambda i: (0, 0)),
            pl.BlockSpec((_BLK, D), lambda i: (i, 0)),
        ],
        out_specs=pl.BlockSpec((_BLK, D), lambda i: (i, 0)),
        out_shape=jax.ShapeDtypeStruct((NC * NP, D), jnp.float32),
    )(xs, W1, b1, deg16)


def _run_k2(agg, W2, b2, deg16):
    return pl.pallas_call(
        _k2_body,
        grid=((NC * NP) // _BLK,),
        in_specs=[
            pl.BlockSpec((_BLK, D), lambda i: (i, 0)),
            pl.BlockSpec((D, D), lambda i: (0, 0)),
            pl.BlockSpec((1, D), lambda i: (0, 0)),
            pl.BlockSpec((_BLK, D), lambda i: (i, 0)),
        ],
        out_specs=pl.BlockSpec((_BLK, D), lambda i: (i, 0)),
        out_shape=jax.ShapeDtypeStruct((NC * NP, D), jnp.float32),
    )(agg, W2, b2, deg16)


def _run_k3(agg, deg16):
    return pl.pallas_call(
        _k3_body,
        grid=((NC * NP) // _BLK,),
        in_specs=[
            pl.BlockSpec((_BLK, D), lambda i: (i, 0)),
            pl.BlockSpec((_BLK, D), lambda i: (i, 0)),
        ],
        out_specs=[
            pl.BlockSpec((_BLK, D), lambda i: (i, 0)),
            pl.BlockSpec((_BLK, D), lambda i: (i, 0)),
        ],
        out_shape=[
            jax.ShapeDtypeStruct((NC * NP, D), jnp.float32),
            jax.ShapeDtypeStruct((NC * NP, D), jnp.float32),
        ],
    )(agg, deg16)


_LB = 1000
_LG = N // _LB


def _loss_body(z1_ref, z2_ref, o_ref, rs_ref, cs_ref, acc_ref):
    i = pl.program_id(0)
    j = pl.program_id(1)
    sim = jax.lax.dot_general(
        z1_ref[...], z2_ref[...], (((1,), (1,)), ((), ())),
        preferred_element_type=jnp.float32) * INV_TAU
    es = jnp.exp(sim)
    row_s = jnp.sum(es, axis=1, keepdims=True)
    col_s = jnp.sum(es, axis=0, keepdims=True)

    @pl.when((i == 0) & (j == 0))
    def _():
        acc_ref[0] = 0.0

    @pl.when(j == 0)
    def _():
        rs_ref[...] = row_s

    @pl.when(j != 0)
    def _():
        rs_ref[...] += row_s

    @pl.when(i == 0)
    def _():
        cs_ref[pl.ds(j, 1), :] = col_s

    @pl.when(i != 0)
    def _():
        cs_ref[pl.ds(j, 1), :] += col_s

    @pl.when(i == j)
    def _():
        r = lax.broadcasted_iota(jnp.int32, (_LB, _LB), 0)
        c = lax.broadcasted_iota(jnp.int32, (_LB, _LB), 1)
        acc_ref[0] += 2.0 * jnp.sum(jnp.where(r == c, sim, 0.0))

    @pl.when(j == _LG - 1)
    def _():
        acc_ref[0] += -jnp.sum(jnp.log(rs_ref[...]))

    @pl.when((i == _LG - 1) & (j == _LG - 1))
    def _():
        total = acc_ref[0] - jnp.sum(jnp.log(cs_ref[...]))
        o_ref[...] = jnp.reshape(-0.5 * total / N, (1, 1))


def _run_loss(z1n, z2n):
    return pl.pallas_call(
        _loss_body,
        grid=(_LG, _LG),
        in_specs=[
            pl.BlockSpec((_LB, D), lambda i, j: (i, 0)),
            pl.BlockSpec((_LB, D), lambda i, j: (j, 0)),
        ],
        out_specs=pl.BlockSpec((1, 1), lambda i, j: (0, 0)),
        out_shape=jax.ShapeDtypeStruct((1, 1), jnp.float32),
        scratch_shapes=[
            pltpu.VMEM((_LB, 1), jnp.float32),
            pltpu.VMEM((_LG, _LB), jnp.float32),
            pltpu.SMEM((1,), jnp.float32),
        ],
    )(z1n, z2n)


def kernel(x1, edge_index1, x2, edge_index2, W1, b1, W2, b2):
    ei1 = edge_index1.astype(jnp.int32)
    ei2 = edge_index2.astype(jnp.int32)
    src_flat = jnp.concatenate([ei1[0], ei2[0]])
    dst_flat = jnp.concatenate([ei1[1], ei2[1]])
    pad = jnp.zeros((NP - N, D), jnp.float32)
    xs = jnp.concatenate([x1, pad, x2, pad], axis=0)
    onesD = jnp.ones((EDGE_CHUNK, D), jnp.float32)
    zerosD = jnp.zeros((NP, D), jnp.float32)
    b1r = b1.reshape(1, D)
    b2r = b2.reshape(1, D)

    deg16 = _deg_call(dst_flat, onesD, zerosD)

    h1s = _run_k1(xs, W1, b1r, deg16)
    agg1 = _agg_call(h1s, src_flat, dst_flat, zerosD)
    h2s = _run_k2(agg1, W2, b2r, deg16)
    agg2 = _agg_call(h2s, src_flat, dst_flat, zerosD)
    z, zn = _run_k3(agg2, deg16)

    z1 = lax.slice(z, (0, 0), (N, D))
    z1n = lax.slice(zn, (0, 0), (N, D))
    z2n = lax.slice(zn, (NP, 0), (NP + N, D))
    loss = _run_loss(z1n, z2n)[0, 0]
    return (z1, loss)
